# baseline (speedup 1.0000x reference)
import jax
import jax.numpy as jnp
from jax.experimental import pallas as pl

N = 65536
IN_FEATURES = 1024
OUT_FEATURES = 16
BLOCK_N = 1024


def _matmul_body(x_ref, w_ref, b_ref, out_ref):
    acc = jax.lax.dot_general(
        x_ref[...],
        w_ref[...],
        dimension_numbers=(((1,), (1,)), ((), ())),
        preferred_element_type=jnp.float32,
    )
    out_ref[...] = (acc + b_ref[...]).T


def kernel(input, weight, bias):
    b_row = bias.reshape(1, OUT_FEATURES)
    out_t = pl.pallas_call(
        _matmul_body,
        grid=(N // BLOCK_N,),
        in_specs=[
            pl.BlockSpec((BLOCK_N, IN_FEATURES), lambda i: (i, 0)),
            pl.BlockSpec((OUT_FEATURES, IN_FEATURES), lambda i: (0, 0)),
            pl.BlockSpec((1, OUT_FEATURES), lambda i: (0, 0)),
        ],
        out_specs=pl.BlockSpec((OUT_FEATURES, BLOCK_N), lambda i: (0, i)),
        out_shape=jax.ShapeDtypeStruct((OUT_FEATURES, N), jnp.float32),
    )(input, weight, b_row)
    return out_t.T


# speedup vs baseline: 1.2200x; 1.2200x over previous
import jax
import jax.numpy as jnp
from jax.experimental import pallas as pl

N = 65536
IN_FEATURES = 1024
OUT_FEATURES = 16
BLOCK_N = 2048


def _matmul_body(x_ref, w_ref, b_ref, out_ref):
    acc = jax.lax.dot_general(
        x_ref[...],
        w_ref[...],
        dimension_numbers=(((1,), (1,)), ((), ())),
        preferred_element_type=jnp.float32,
    )
    out_ref[...] = (acc + b_ref[...]).T


def kernel(input, weight, bias):
    b_row = bias.reshape(1, OUT_FEATURES)
    out_t = pl.pallas_call(
        _matmul_body,
        grid=(N // BLOCK_N,),
        in_specs=[
            pl.BlockSpec((BLOCK_N, IN_FEATURES), lambda i: (i, 0)),
            pl.BlockSpec((OUT_FEATURES, IN_FEATURES), lambda i: (0, 0)),
            pl.BlockSpec((1, OUT_FEATURES), lambda i: (0, 0)),
        ],
        out_specs=pl.BlockSpec((OUT_FEATURES, BLOCK_N), lambda i: (0, i)),
        out_shape=jax.ShapeDtypeStruct((OUT_FEATURES, N), jnp.float32),
    )(input, weight, b_row)
    return out_t.T
